# baseline (speedup 1.0000x reference)
import functools

import jax
import jax.numpy as jnp
from jax import lax
from jax.experimental import pallas as pl
from jax.experimental.pallas import tpu as pltpu
from jax.experimental.pallas import tpu_sc as plsc

N = 100000
D = 128
H = 64
S = 1024

NC = 2
NS = 16
NW = NC * NS

CHUNK = 240
FULL_CHUNKS = 13
VPC = CHUNK // 16
TAIL = 8
ROWS_PER_STRIPE = S // NS
PAD = 8
IDXH_LEN = PAD + FULL_CHUNKS * CHUNK + TAIL + 16
BIG = 1 << 30


def _sc_body(scaler_hbm, bidx_hbm, sums_out, firsts_out, lasts_out,
             rows_v, rows8_v, idxh, first_v, last_v,
             stage_v, acc_sh, ld_sem, st_sem, hx_sem):
    c = lax.axis_index("c")
    s = lax.axis_index("s")
    w = s * NC + c

    base = 8 * (390 * w + jnp.minimum(w, 20))
    nr = jnp.where(w < 20, FULL_CHUNKS * CHUNK + TAIL, FULL_CHUNKS * CHUNK)
    lane = lax.iota(jnp.int32, 16)

    hx = pltpu.async_copy(bidx_hbm.at[pl.ds(base, FULL_CHUNKS * CHUNK)],
                          idxh.at[pl.ds(PAD, FULL_CHUNKS * CHUNK)], hx_sem)

    @pl.when(w > 0)
    def _prev8():
        pltpu.sync_copy(bidx_hbm.at[pl.ds(base - PAD, PAD)], idxh.at[pl.ds(0, PAD)])

    @pl.when(w < NW - 1)
    def _next8():
        pltpu.sync_copy(bidx_hbm.at[pl.ds(base + nr, 8)],
                        idxh.at[pl.ds(PAD + nr, 8)])

    @pl.when(w < 20)
    def _tail_idx():
        pltpu.sync_copy(bidx_hbm.at[pl.ds(base + FULL_CHUNKS * CHUNK, TAIL)],
                        idxh.at[pl.ds(PAD + FULL_CHUNKS * CHUNK, TAIL)])

    zero16 = jnp.zeros((16,), jnp.float32)
    big16 = jnp.full((16,), BIG, jnp.int32)
    neg16 = jnp.full((16,), -1, jnp.int32)
    for r in range(ROWS_PER_STRIPE):
        for g in range(D // 16):
            stage_v[r, pl.ds(g * 16, 16)] = zero16
    for g in range(S // 16):
        first_v[pl.ds(g * 16, 16)] = big16
        last_v[pl.ds(g * 16, 16)] = neg16

    pltpu.sync_copy(stage_v, acc_sh.at[pl.ds(s * ROWS_PER_STRIPE, ROWS_PER_STRIPE)])

    hx.wait()

    @pl.when(w == 0)
    def _first_sentinel():
        plsc.store_scatter(idxh, [lane + (PAD - 1)], jnp.full((16,), -1, jnp.int32),
                           mask=lane == 0)

    @pl.when(w == NW - 1)
    def _last_sentinel():
        plsc.store_scatter(idxh, [lane + PAD + FULL_CHUNKS * CHUNK],
                           jnp.full((16,), S, jnp.int32), mask=lane == 0)

    plsc.subcore_barrier()

    def scan_vreg(j, valid8):
        off = PAD + 16 * j
        xp = idxh[pl.ds(off - 1, 16)]
        x = idxh[pl.ds(off, 16)]
        xn = idxh[pl.ds(off + 1, 16)]
        pos = (base + 16 * j) + lane
        mf = x != xp
        ml = x != xn
        if valid8:
            mf = mf & (lane < 8)
            ml = ml & (lane < 8)
        plsc.store_scatter(first_v, [x], pos, mask=mf)
        plsc.store_scatter(last_v, [x], pos, mask=ml)

    def start_load(i, b):
        r0 = base + i * CHUNK
        return pltpu.async_copy(scaler_hbm.at[pl.ds(r0, CHUNK)], rows_v.at[b],
                                ld_sem[b])

    ld_desc = [None, None]
    st_desc = [None, None]
    ld_desc[0] = start_load(0, 0)
    for i in range(FULL_CHUNKS):
        b = i % 2
        o = 1 - b
        if i >= 1:
            st_desc[o].wait()
        if i + 1 < FULL_CHUNKS:
            ld_desc[o] = start_load(i + 1, o)
        ld_desc[b].wait()
        st_desc[b] = pltpu.async_copy(
            rows_v.at[b], acc_sh.at[idxh.at[pl.ds(PAD + i * CHUNK, CHUNK)]],
            st_sem[b], add=True)
        for v in range(VPC):
            scan_vreg(i * VPC + v, False)
    st_desc[(FULL_CHUNKS - 1) % 2].wait()

    @pl.when(w < 20)
    def _tail():
        r0 = base + FULL_CHUNKS * CHUNK
        pltpu.sync_copy(scaler_hbm.at[pl.ds(r0, TAIL)], rows8_v)
        pltpu.sync_copy(rows8_v,
                        acc_sh.at[idxh.at[pl.ds(PAD + FULL_CHUNKS * CHUNK, TAIL)]],
                        add=True)
        scan_vreg(FULL_CHUNKS * VPC, True)

    plsc.subcore_barrier()

    row = c * S + s * ROWS_PER_STRIPE
    pltpu.sync_copy(acc_sh.at[pl.ds(s * ROWS_PER_STRIPE, ROWS_PER_STRIPE)], stage_v)
    pltpu.sync_copy(stage_v, sums_out.at[pl.ds(row, ROWS_PER_STRIPE)])
    pltpu.sync_copy(first_v, firsts_out.at[w])
    pltpu.sync_copy(last_v, lasts_out.at[w])


def _sc_kernel_fn(scaler_hbm, bidx_hbm, sums_out, firsts_out, lasts_out, *scratch):
    _sc_body(scaler_hbm, bidx_hbm, sums_out, firsts_out, lasts_out, *scratch)


@functools.cache
def _make_sc_segment_sum():
  return pl.kernel(
    _sc_kernel_fn,
    out_type=(
        jax.ShapeDtypeStruct((NC * S, D), jnp.float32),
        jax.ShapeDtypeStruct((NW, S), jnp.int32),
        jax.ShapeDtypeStruct((NW, S), jnp.int32),
    ),
    mesh=plsc.VectorSubcoreMesh(core_axis_name="c", subcore_axis_name="s",
                                num_cores=NC, num_subcores=NS),
    compiler_params=pltpu.CompilerParams(needs_layout_passes=False),
    scratch_types=[
        pltpu.VMEM((2, CHUNK, D), jnp.float32),
        pltpu.VMEM((TAIL, D), jnp.float32),
        pltpu.VMEM((IDXH_LEN,), jnp.int32),
        pltpu.VMEM((S,), jnp.int32),
        pltpu.VMEM((S,), jnp.int32),
        pltpu.VMEM((ROWS_PER_STRIPE, D), jnp.float32),
        pltpu.VMEM_SHARED((S, D), jnp.float32),
        (pltpu.SemaphoreType.DMA, pltpu.SemaphoreType.DMA),
        (pltpu.SemaphoreType.DMA, pltpu.SemaphoreType.DMA),
        pltpu.SemaphoreType.DMA,
    ],
  )


def _mlp_body(sums_ref, firsts_ref, lasts_ref, w1_ref, b1_ref, w2t_ref, b2_ref,
              out_ref):
    sums = sums_ref[0:S, :] + sums_ref[S:2 * S, :]
    fmin = jnp.min(firsts_ref[...], axis=0)
    lmax = jnp.max(lasts_ref[...], axis=0)
    cnt = (lmax - fmin + 1).astype(jnp.float32)
    graph = sums / jnp.maximum(cnt, 1.0)[:, None]
    h = lax.dot_general(graph, w1_ref[...], (((1,), (0,)), ((), ())),
                        precision=lax.Precision.HIGHEST,
                        preferred_element_type=jnp.float32)
    h = jnp.maximum(h + b1_ref[...], 0.0)
    out = jnp.sum(h * w2t_ref[...], axis=1, keepdims=True) + b2_ref[0, 0]
    out_ref[...] = out


def _mlp_head(sums_p, firsts_p, lasts_p, W1, b1, W2, b2):
    return pl.pallas_call(
        _mlp_body,
        out_shape=jax.ShapeDtypeStruct((S, 1), jnp.float32),
    )(sums_p, firsts_p, lasts_p, W1, b1.reshape(1, H), W2.reshape(1, H),
      b2.reshape(1, 1))


def kernel(pos, scaler, vector, batch_index, W1, b1, W2, b2):
    sums_p, firsts_p, lasts_p = _make_sc_segment_sum()(scaler, batch_index)
    return _mlp_head(sums_p, firsts_p, lasts_p, W1, b1, W2, b2)


# speedup vs baseline: 1.0515x; 1.0515x over previous
import functools

import jax
import jax.numpy as jnp
from jax import lax
from jax.experimental import pallas as pl
from jax.experimental.pallas import tpu as pltpu
from jax.experimental.pallas import tpu_sc as plsc

N = 100000
D = 128
H = 64
S = 1024

NC = 2
NS = 16
NW = NC * NS

CHUNK = 80
FULL_CHUNKS = 39
NBUF = 6
LOOKAHEAD = 2
VPC = CHUNK // 16
TAIL = 8
ROWS_PER_STRIPE = S // NS
PAD = 8
IDXH_LEN = PAD + FULL_CHUNKS * CHUNK + TAIL + 16
BIG = 1 << 30


def _sc_body(scaler_hbm, bidx_hbm, sums_out, firsts_out, lasts_out,
             rows_v, rows8_v, idxh, first_v, last_v,
             stage_v, acc_sh, ld_sem, st_sem, hx_sem):
    c = lax.axis_index("c")
    s = lax.axis_index("s")
    w = s * NC + c

    base = 8 * (390 * w + jnp.minimum(w, 20))
    nr = jnp.where(w < 20, FULL_CHUNKS * CHUNK + TAIL, FULL_CHUNKS * CHUNK)
    lane = lax.iota(jnp.int32, 16)

    hx = pltpu.async_copy(bidx_hbm.at[pl.ds(base, FULL_CHUNKS * CHUNK)],
                          idxh.at[pl.ds(PAD, FULL_CHUNKS * CHUNK)], hx_sem)

    @pl.when(w > 0)
    def _prev8():
        pltpu.sync_copy(bidx_hbm.at[pl.ds(base - PAD, PAD)], idxh.at[pl.ds(0, PAD)])

    @pl.when(w < NW - 1)
    def _next8():
        pltpu.sync_copy(bidx_hbm.at[pl.ds(base + nr, 8)],
                        idxh.at[pl.ds(PAD + nr, 8)])

    @pl.when(w < 20)
    def _tail_idx():
        pltpu.sync_copy(bidx_hbm.at[pl.ds(base + FULL_CHUNKS * CHUNK, TAIL)],
                        idxh.at[pl.ds(PAD + FULL_CHUNKS * CHUNK, TAIL)])

    zero16 = jnp.zeros((16,), jnp.float32)
    big16 = jnp.full((16,), BIG, jnp.int32)
    neg16 = jnp.full((16,), -1, jnp.int32)
    for r in range(ROWS_PER_STRIPE):
        for g in range(D // 16):
            stage_v[r, pl.ds(g * 16, 16)] = zero16
    for g in range(S // 16):
        first_v[pl.ds(g * 16, 16)] = big16
        last_v[pl.ds(g * 16, 16)] = neg16

    pltpu.sync_copy(stage_v, acc_sh.at[pl.ds(s * ROWS_PER_STRIPE, ROWS_PER_STRIPE)])

    hx.wait()

    @pl.when(w == 0)
    def _first_sentinel():
        plsc.store_scatter(idxh, [lane + (PAD - 1)], jnp.full((16,), -1, jnp.int32),
                           mask=lane == 0)

    @pl.when(w == NW - 1)
    def _last_sentinel():
        plsc.store_scatter(idxh, [lane + PAD + FULL_CHUNKS * CHUNK],
                           jnp.full((16,), S, jnp.int32), mask=lane == 0)

    plsc.subcore_barrier()

    def scan_vreg(j, valid8):
        off = PAD + 16 * j
        xp = idxh[pl.ds(off - 1, 16)]
        x = idxh[pl.ds(off, 16)]
        xn = idxh[pl.ds(off + 1, 16)]
        pos = (base + 16 * j) + lane
        mf = x != xp
        ml = x != xn
        if valid8:
            mf = mf & (lane < 8)
            ml = ml & (lane < 8)
        plsc.store_scatter(first_v, [x], pos, mask=mf)
        plsc.store_scatter(last_v, [x], pos, mask=ml)

    def start_load(i):
        r0 = base + i * CHUNK
        return pltpu.async_copy(scaler_hbm.at[pl.ds(r0, CHUNK)],
                                rows_v.at[i % NBUF], ld_sem[i % NBUF])

    ld_desc = [None] * FULL_CHUNKS
    st_desc = [None] * FULL_CHUNKS
    for j in range(LOOKAHEAD):
        ld_desc[j] = start_load(j)
    for i in range(FULL_CHUNKS):
        j = i + LOOKAHEAD
        if j < FULL_CHUNKS:
            if j - NBUF >= 0:
                st_desc[j - NBUF].wait()
            ld_desc[j] = start_load(j)
        ld_desc[i].wait()
        st_desc[i] = pltpu.async_copy(
            rows_v.at[i % NBUF],
            acc_sh.at[idxh.at[pl.ds(PAD + i * CHUNK, CHUNK)]],
            st_sem[i % NBUF], add=True)
        for v in range(VPC):
            scan_vreg(i * VPC + v, False)
    for i in range(max(0, FULL_CHUNKS - NBUF), FULL_CHUNKS):
        st_desc[i].wait()

    @pl.when(w < 20)
    def _tail():
        r0 = base + FULL_CHUNKS * CHUNK
        pltpu.sync_copy(scaler_hbm.at[pl.ds(r0, TAIL)], rows8_v)
        pltpu.sync_copy(rows8_v,
                        acc_sh.at[idxh.at[pl.ds(PAD + FULL_CHUNKS * CHUNK, TAIL)]],
                        add=True)
        scan_vreg(FULL_CHUNKS * VPC, True)

    plsc.subcore_barrier()

    row = c * S + s * ROWS_PER_STRIPE
    pltpu.sync_copy(acc_sh.at[pl.ds(s * ROWS_PER_STRIPE, ROWS_PER_STRIPE)], stage_v)
    pltpu.sync_copy(stage_v, sums_out.at[pl.ds(row, ROWS_PER_STRIPE)])
    pltpu.sync_copy(first_v, firsts_out.at[w])
    pltpu.sync_copy(last_v, lasts_out.at[w])


def _sc_kernel_fn(scaler_hbm, bidx_hbm, sums_out, firsts_out, lasts_out, *scratch):
    _sc_body(scaler_hbm, bidx_hbm, sums_out, firsts_out, lasts_out, *scratch)


@functools.cache
def _make_sc_segment_sum():
  return pl.kernel(
    _sc_kernel_fn,
    out_type=(
        jax.ShapeDtypeStruct((NC * S, D), jnp.float32),
        jax.ShapeDtypeStruct((NW, S), jnp.int32),
        jax.ShapeDtypeStruct((NW, S), jnp.int32),
    ),
    mesh=plsc.VectorSubcoreMesh(core_axis_name="c", subcore_axis_name="s",
                                num_cores=NC, num_subcores=NS),
    compiler_params=pltpu.CompilerParams(needs_layout_passes=False),
    scratch_types=[
        pltpu.VMEM((NBUF, CHUNK, D), jnp.float32),
        pltpu.VMEM((TAIL, D), jnp.float32),
        pltpu.VMEM((IDXH_LEN,), jnp.int32),
        pltpu.VMEM((S,), jnp.int32),
        pltpu.VMEM((S,), jnp.int32),
        pltpu.VMEM((ROWS_PER_STRIPE, D), jnp.float32),
        pltpu.VMEM_SHARED((S, D), jnp.float32),
        tuple(pltpu.SemaphoreType.DMA for _ in range(NBUF)),
        tuple(pltpu.SemaphoreType.DMA for _ in range(NBUF)),
        pltpu.SemaphoreType.DMA,
    ],
  )


def _mlp_body(sums_ref, firsts_ref, lasts_ref, w1_ref, b1_ref, w2t_ref, b2_ref,
              out_ref):
    sums = sums_ref[0:S, :] + sums_ref[S:2 * S, :]
    fmin = jnp.min(firsts_ref[...], axis=0)
    lmax = jnp.max(lasts_ref[...], axis=0)
    cnt = (lmax - fmin + 1).astype(jnp.float32)
    graph = sums / jnp.maximum(cnt, 1.0)[:, None]
    h = lax.dot_general(graph, w1_ref[...], (((1,), (0,)), ((), ())),
                        precision=lax.Precision.HIGHEST,
                        preferred_element_type=jnp.float32)
    h = jnp.maximum(h + b1_ref[...], 0.0)
    out = jnp.sum(h * w2t_ref[...], axis=1, keepdims=True) + b2_ref[0, 0]
    out_ref[...] = out


def _mlp_head(sums_p, firsts_p, lasts_p, W1, b1, W2, b2):
    return pl.pallas_call(
        _mlp_body,
        out_shape=jax.ShapeDtypeStruct((S, 1), jnp.float32),
    )(sums_p, firsts_p, lasts_p, W1, b1.reshape(1, H), W2.reshape(1, H),
      b2.reshape(1, 1))


def kernel(pos, scaler, vector, batch_index, W1, b1, W2, b2):
    sums_p, firsts_p, lasts_p = _make_sc_segment_sum()(scaler, batch_index)
    return _mlp_head(sums_p, firsts_p, lasts_p, W1, b1, W2, b2)
